# Initial kernel scaffold; baseline (speedup 1.0000x reference)
import jax
import jax.numpy as jnp
from jax.experimental import pallas as pl


def kernel(xyz, surf_dirs, conv_w, conv_b, conv_dirs, fc1_w, fc1_b, fc2_w, fc2_b, d1_w, d1_b, d2_w, d2_b, g1_w, g1_b, g2_w, g2_b, wq, wk, wv):
    raise NotImplementedError("write your pallas kernel here")


# speedup vs baseline: 10.1239x; 10.1239x over previous
import functools

import numpy as np
import jax
import jax.numpy as jnp
from jax import lax
from jax.experimental import pallas as pl
from jax.experimental.pallas import tpu as pltpu
from jax.experimental.pallas import tpu_sc as plsc

_BS, _N, _NN, _AK, _C = 2, 4096, 20, 16, 128
_K21 = _NN + 1
_R = 256
_CH = 128
_NW = 32
_VD = 128


def _topk_body(vb_ref, va_ref, out_ref, d_scr):
    b = pl.program_id(0)
    vb = vb_ref[0]
    va = va_ref[0]
    inner = lax.dot_general(vb, va, (((1,), (1,)), ((), ())),
                            preferred_element_type=jnp.float32)
    qb = jnp.sum(vb * vb, axis=1)
    qa = jnp.sum(va * va, axis=1)
    d_scr[...] = -2.0 * inner + qb[:, None] + qa[None, :]
    cols = lax.broadcasted_iota(jnp.int32, (_R, _N), 1)
    k_iota = lax.broadcasted_iota(jnp.int32, (_R, _K21), 1)

    def body(i, idxacc):
        dmat = d_scr[...]
        m = jnp.min(dmat, axis=1, keepdims=True)
        sel = jnp.where(dmat == m, cols, jnp.int32(2**31 - 1))
        j = jnp.min(sel, axis=1, keepdims=True)
        d_scr[...] = jnp.where(cols == j, jnp.float32(jnp.inf), dmat)
        return jnp.where(k_iota == i, j, idxacc)

    idxacc = lax.fori_loop(0, _K21, body, jnp.zeros((_R, _K21), jnp.int32))
    out_ref[0] = idxacc + b * _N


def _topk(v):
    return pl.pallas_call(
        _topk_body,
        grid=(_BS, _N // _R),
        in_specs=[pl.BlockSpec((1, _R, 3), lambda b, r: (b, r, 0)),
                  pl.BlockSpec((1, _N, 3), lambda b, r: (b, 0, 0))],
        out_specs=pl.BlockSpec((1, _R, _K21), lambda b, r: (b, r, 0)),
        out_shape=jax.ShapeDtypeStruct((_BS, _N, _K21), jnp.int32),
        scratch_shapes=[pltpu.VMEM((_R, _N), jnp.float32)],
    )(v, v)


def _sc_gather(table, idx):
    m_total = idx.shape[0]
    d = table.shape[1]
    nchunk = m_total // (_NW * _CH)
    idx2 = idx.reshape(_NW, nchunk * _CH)
    mesh = plsc.VectorSubcoreMesh(core_axis_name="c", subcore_axis_name="s")

    @functools.partial(
        pl.kernel,
        out_type=jax.ShapeDtypeStruct((m_total, d), jnp.float32),
        mesh=mesh,
        scratch_types=[
            pltpu.VMEM((nchunk * _CH,), jnp.int32),
            pltpu.VMEM((_CH, d), jnp.float32),
            pltpu.SemaphoreType.DMA,
        ],
    )
    def k(table_hbm, idx_hbm, out_hbm, idx_v, buf_v, sem):
        wid = lax.axis_index("s") * 2 + lax.axis_index("c")
        pltpu.sync_copy(idx_hbm.at[wid], idx_v)
        base = wid * (nchunk * _CH)

        def body(j, carry):
            rows = idx_v.at[pl.ds(j * _CH, _CH)]
            pltpu.async_copy(table_hbm.at[rows], buf_v, sem).wait()
            pltpu.sync_copy(buf_v, out_hbm.at[pl.ds(base + j * _CH, _CH)])
            return carry

        lax.fori_loop(0, nchunk, body, 0)

    return k(table, idx2)


def _dirn_from(vnbr_ref, vb_ref):
    vn = vnbr_ref[0].reshape(_R, _K21, _VD)[:, 1:, :3]
    vb = vb_ref[0]
    diff = vn - vb[:, None, :]
    nrm = jnp.sqrt(jnp.sum(diff * diff, axis=-1, keepdims=True))
    return (diff / jnp.maximum(nrm, 1e-12)).reshape(_R * _NN, 3)


def _coldir(w_ref):
    w = w_ref[...]
    nrm = jnp.sqrt(jnp.sum(w * w, axis=0, keepdims=True))
    return w / jnp.maximum(nrm, 1e-12)


def _mm(a, b):
    return lax.dot_general(a, b, (((1,), (0,)), ((), ())),
                           preferred_element_type=jnp.float32)


def _s2_body(vnbr_ref, vb_ref, sd_ref, cw_ref, cb_ref, cen_ref, sup_ref):
    dirn = _dirn_from(vnbr_ref, vb_ref)
    theta = jnp.maximum(_mm(dirn, _coldir(sd_ref)), 0.0)
    fea = jnp.maximum(jnp.max(theta.reshape(_R, _NN, _C), axis=1), 0.0)
    fout = _mm(fea, cw_ref[...]) + cb_ref[...]
    cen_ref[0] = fout[:, :_C]
    sup_ref[0] = fout[:, _C:]


def _s2(vnbr, v, surf_dirs, conv_w, conv_b):
    return pl.pallas_call(
        _s2_body,
        grid=(_BS, _N // _R),
        in_specs=[
            pl.BlockSpec((1, _R * _K21, _VD), lambda b, r: (b, r, 0)),
            pl.BlockSpec((1, _R, 3), lambda b, r: (b, r, 0)),
            pl.BlockSpec((3, _C), lambda b, r: (0, 0)),
            pl.BlockSpec((_C, 2 * _C), lambda b, r: (0, 0)),
            pl.BlockSpec((1, 2 * _C), lambda b, r: (0, 0)),
        ],
        out_specs=[pl.BlockSpec((1, _R, _C), lambda b, r: (b, r, 0)),
                   pl.BlockSpec((1, _R, _C), lambda b, r: (b, r, 0))],
        out_shape=[jax.ShapeDtypeStruct((_BS, _N, _C), jnp.float32),
                   jax.ShapeDtypeStruct((_BS, _N, _C), jnp.float32)],
    )(vnbr, v, surf_dirs, conv_w, conv_b)


def _s3_body(vnbr_ref, vb_ref, cd_ref, cen_ref, supg_ref,
             f1w_ref, f1b_ref, wq_ref, wk_ref, wv_ref,
             pre_ref, q_ref, kv_ref):
    dirn = _dirn_from(vnbr_ref, vb_ref)
    theta2 = jnp.maximum(_mm(dirn, _coldir(cd_ref)), 0.0)
    act = theta2.reshape(_R, _NN, _C) * supg_ref[0].reshape(_R, _NN, _C)
    pre = cen_ref[0] + jnp.max(act, axis=1)
    x = _mm(pre, f1w_ref[...]) + f1b_ref[...]
    pre_ref[0] = pre
    q_ref[0] = _mm(x, wq_ref[...])
    kv_ref[0] = jnp.concatenate([_mm(x, wk_ref[...]), _mm(x, wv_ref[...])],
                                axis=1)


def _s3(vnbr, v, conv_dirs, center, supg, fc1_w, fc1_b, wq, wk, wv):
    return pl.pallas_call(
        _s3_body,
        grid=(_BS, _N // _R),
        in_specs=[
            pl.BlockSpec((1, _R * _K21, _VD), lambda b, r: (b, r, 0)),
            pl.BlockSpec((1, _R, 3), lambda b, r: (b, r, 0)),
            pl.BlockSpec((3, _C), lambda b, r: (0, 0)),
            pl.BlockSpec((1, _R, _C), lambda b, r: (b, r, 0)),
            pl.BlockSpec((1, _R * _NN, _C), lambda b, r: (b, r, 0)),
            pl.BlockSpec((_C, _C), lambda b, r: (0, 0)),
            pl.BlockSpec((1, _C), lambda b, r: (0, 0)),
            pl.BlockSpec((_C, _C), lambda b, r: (0, 0)),
            pl.BlockSpec((_C, _C), lambda b, r: (0, 0)),
            pl.BlockSpec((_C, _C), lambda b, r: (0, 0)),
        ],
        out_specs=[pl.BlockSpec((1, _R, _C), lambda b, r: (b, r, 0)),
                   pl.BlockSpec((1, _R, _C), lambda b, r: (b, r, 0)),
                   pl.BlockSpec((1, _R, 2 * _C), lambda b, r: (b, r, 0))],
        out_shape=[jax.ShapeDtypeStruct((_BS, _N, _C), jnp.float32),
                   jax.ShapeDtypeStruct((_BS, _N, _C), jnp.float32),
                   jax.ShapeDtypeStruct((_BS, _N, 2 * _C), jnp.float32)],
    )(vnbr, v, conv_dirs, center, supg, fc1_w, fc1_b, wq, wk, wv)


def _s4_body(vnbr_ref, vb_ref, q_ref, kvg_ref, pre_ref,
             d1w_ref, d1b_ref, d2w_ref, d2b_ref,
             g1w_ref, g1b_ref, g2w_ref, g2b_ref,
             f2w_ref, f2b_ref, out_ref):
    vknn = vnbr_ref[0].reshape(_R, _K21, _VD)[:, :_AK, :3]
    vb = vb_ref[0]
    delta = (vb[:, None, :] - vknn).reshape(_R * _AK, 3)
    pos = _mm(jnp.maximum(_mm(delta, d1w_ref[...]) + d1b_ref[...], 0.0),
              d2w_ref[...]) + d2b_ref[...]
    kvg = kvg_ref[0].reshape(_R, _AK, 2 * _C)
    kk = kvg[:, :, :_C].reshape(_R * _AK, _C)
    vv = kvg[:, :, _C:].reshape(_R * _AK, _C)
    qb = jnp.broadcast_to(q_ref[0][:, None, :], (_R, _AK, _C))
    t = qb.reshape(_R * _AK, _C) - kk + pos
    attn = _mm(jnp.maximum(_mm(t, g1w_ref[...]) + g1b_ref[...], 0.0),
               g2w_ref[...]) + g2b_ref[...]
    attn = (attn * np.float32(1.0 / np.sqrt(_C))).reshape(_R, _AK, _C)
    attn = attn - jnp.max(attn, axis=1, keepdims=True)
    e = jnp.exp(attn)
    w = e / jnp.sum(e, axis=1, keepdims=True)
    res = jnp.sum(w * (vv + pos).reshape(_R, _AK, _C), axis=1)
    out_ref[0] = _mm(res, f2w_ref[...]) + f2b_ref[...] + pre_ref[0]


def _s4(vnbr, v, q, kvg, pre, d1_w, d1_b, d2_w, d2_b,
        g1_w, g1_b, g2_w, g2_b, fc2_w, fc2_b):
    wspec = lambda shape: pl.BlockSpec(shape, lambda b, r: tuple(0 for _ in shape))
    return pl.pallas_call(
        _s4_body,
        grid=(_BS, _N // _R),
        in_specs=[
            pl.BlockSpec((1, _R * _K21, _VD), lambda b, r: (b, r, 0)),
            pl.BlockSpec((1, _R, 3), lambda b, r: (b, r, 0)),
            pl.BlockSpec((1, _R, _C), lambda b, r: (b, r, 0)),
            pl.BlockSpec((1, _R * _AK, 2 * _C), lambda b, r: (b, r, 0)),
            pl.BlockSpec((1, _R, _C), lambda b, r: (b, r, 0)),
            wspec((3, _C)), wspec((1, _C)), wspec((_C, _C)), wspec((1, _C)),
            wspec((_C, _C)), wspec((1, _C)), wspec((_C, _C)), wspec((1, _C)),
            wspec((_C, _C)), wspec((1, _C)),
        ],
        out_specs=pl.BlockSpec((1, _R, _C), lambda b, r: (b, r, 0)),
        out_shape=jax.ShapeDtypeStruct((_BS, _N, _C), jnp.float32),
    )(vnbr, v, q, kvg, pre, d1_w, d1_b, d2_w, d2_b,
      g1_w, g1_b, g2_w, g2_b, fc2_w, fc2_b)


def kernel(xyz, surf_dirs, conv_w, conv_b, conv_dirs, fc1_w, fc1_b, fc2_w,
           fc2_b, d1_w, d1_b, d2_w, d2_b, g1_w, g1_b, g2_w, g2_b, wq, wk, wv):
    v = jnp.transpose(xyz, (0, 2, 1))
    gidx = _topk(v)

    vpad = jnp.pad(v, ((0, 0), (0, 0), (0, _VD - 3))).reshape(_BS * _N, _VD)
    vnbr = _sc_gather(vpad, gidx.reshape(-1)).reshape(_BS, _N * _K21, _VD)

    center, support = _s2(vnbr, v, surf_dirs, conv_w,
                          conv_b.reshape(1, 2 * _C))

    supg = _sc_gather(support.reshape(_BS * _N, _C),
                      gidx[:, :, 1:].reshape(-1)).reshape(_BS, _N * _NN, _C)

    pre, q, kv = _s3(vnbr, v, conv_dirs, center, supg,
                     fc1_w, fc1_b.reshape(1, _C), wq, wk, wv)

    kvg = _sc_gather(kv.reshape(_BS * _N, 2 * _C),
                     gidx[:, :, :_AK].reshape(-1)).reshape(_BS, _N * _AK, 2 * _C)

    res = _s4(vnbr, v, q, kvg, pre,
              d1_w, d1_b.reshape(1, _C), d2_w, d2_b.reshape(1, _C),
              g1_w, g1_b.reshape(1, _C), g2_w, g2_b.reshape(1, _C),
              fc2_w, fc2_b.reshape(1, _C))
    return jnp.transpose(res, (0, 2, 1))


# speedup vs baseline: 13.8721x; 1.3702x over previous
import functools

import numpy as np
import jax
import jax.numpy as jnp
from jax import lax
from jax.experimental import pallas as pl
from jax.experimental.pallas import tpu as pltpu
from jax.experimental.pallas import tpu_sc as plsc

_BS, _N, _NN, _AK, _C = 2, 4096, 20, 16, 128
_K21 = _NN + 1
_R = 256
_CH = 128
_NW = 32
_VD = 128


def _topk_body(vb_ref, va_ref, out_ref, d_scr):
    b = pl.program_id(0)
    vb = vb_ref[0]
    va = va_ref[0]
    inner = lax.dot_general(vb, va, (((1,), (1,)), ((), ())),
                            preferred_element_type=jnp.float32)
    qb = jnp.sum(vb * vb, axis=1)
    qa = jnp.sum(va * va, axis=1)
    d_scr[...] = -2.0 * inner + qb[:, None] + qa[None, :]
    cols = lax.broadcasted_iota(jnp.int32, (_R, _N), 1)
    k_iota = lax.broadcasted_iota(jnp.int32, (_R, _K21), 1)

    def body(i, idxacc):
        dmat = d_scr[...]
        m = jnp.min(dmat, axis=1, keepdims=True)
        sel = jnp.where(dmat == m, cols, jnp.int32(2**31 - 1))
        j = jnp.min(sel, axis=1, keepdims=True)
        d_scr[...] = jnp.where(cols == j, jnp.float32(jnp.inf), dmat)
        return jnp.where(k_iota == i, j, idxacc)

    idxacc = lax.fori_loop(0, _K21, body, jnp.zeros((_R, _K21), jnp.int32))
    out_ref[0] = idxacc + b * _N


def _topk(vpad3):
    return pl.pallas_call(
        _topk_body,
        grid=(_BS, _N // _R),
        in_specs=[pl.BlockSpec((1, _R, _VD), lambda b, r: (b, r, 0)),
                  pl.BlockSpec((1, _N, _VD), lambda b, r: (b, 0, 0))],
        out_specs=pl.BlockSpec((1, _R, _K21), lambda b, r: (b, r, 0)),
        out_shape=jax.ShapeDtypeStruct((_BS, _N, _K21), jnp.int32),
        scratch_shapes=[pltpu.VMEM((_R, _N), jnp.float32)],
    )(vpad3, vpad3)


def _sc_gather(table, idx):
    m_total = idx.shape[0]
    d = table.shape[1]
    nchunk = m_total // (_NW * _CH)
    idx2 = idx.reshape(_NW, nchunk * _CH)
    mesh = plsc.VectorSubcoreMesh(core_axis_name="c", subcore_axis_name="s")

    @functools.partial(
        pl.kernel,
        out_type=jax.ShapeDtypeStruct((m_total, d), jnp.float32),
        mesh=mesh,
        scratch_types=[
            pltpu.VMEM((nchunk * _CH,), jnp.int32),
            pltpu.VMEM((_CH, d), jnp.float32),
            pltpu.SemaphoreType.DMA,
        ],
    )
    def k(table_hbm, idx_hbm, out_hbm, idx_v, buf_v, sem):
        wid = lax.axis_index("s") * 2 + lax.axis_index("c")
        pltpu.sync_copy(idx_hbm.at[wid], idx_v)
        base = wid * (nchunk * _CH)

        def body(j, carry):
            rows = idx_v.at[pl.ds(j * _CH, _CH)]
            pltpu.async_copy(table_hbm.at[rows], buf_v, sem).wait()
            pltpu.sync_copy(buf_v, out_hbm.at[pl.ds(base + j * _CH, _CH)])
            return carry

        lax.fori_loop(0, nchunk, body, 0)

    return k(table, idx2)


def _dirn_from(vnbr_ref, vb_ref):
    vn = vnbr_ref[0][1:]
    vb = vb_ref[0]
    diff = (vn - vb[None, :, :]).reshape(_NN * _R, _VD)
    ss = _mm(diff * diff, jnp.ones((_VD, _VD), jnp.float32))
    return diff / jnp.maximum(jnp.sqrt(ss), 1e-12)


def _coldir(w_ref):
    w = w_ref[...]
    nrm = jnp.sqrt(jnp.sum(w * w, axis=0, keepdims=True))
    return w / jnp.maximum(nrm, 1e-12)


def _mm(a, b):
    return lax.dot_general(a, b, (((1,), (0,)), ((), ())),
                           preferred_element_type=jnp.float32)


def _s2_body(vnbr_ref, vb_ref, sd_ref, cw_ref, cb_ref, cen_ref, sup_ref):
    dirn = _dirn_from(vnbr_ref, vb_ref)
    theta = jnp.maximum(_mm(dirn, _coldir(sd_ref)), 0.0)
    fea = jnp.maximum(jnp.max(theta.reshape(_NN, _R, _C), axis=0), 0.0)
    fout = _mm(fea, cw_ref[...]) + cb_ref[...]
    cen_ref[0] = fout[:, :_C]
    sup_ref[0] = fout[:, _C:]


def _s2(vnbr, v, surf_dirs, conv_w, conv_b):
    return pl.pallas_call(
        _s2_body,
        grid=(_BS, _N // _R),
        in_specs=[
            pl.BlockSpec((1, _K21, _R, _VD), lambda b, r: (b, 0, r, 0)),
            pl.BlockSpec((1, _R, _VD), lambda b, r: (b, r, 0)),
            pl.BlockSpec((_VD, _C), lambda b, r: (0, 0)),
            pl.BlockSpec((_C, 2 * _C), lambda b, r: (0, 0)),
            pl.BlockSpec((1, 2 * _C), lambda b, r: (0, 0)),
        ],
        out_specs=[pl.BlockSpec((1, _R, _C), lambda b, r: (b, r, 0)),
                   pl.BlockSpec((1, _R, _C), lambda b, r: (b, r, 0))],
        out_shape=[jax.ShapeDtypeStruct((_BS, _N, _C), jnp.float32),
                   jax.ShapeDtypeStruct((_BS, _N, _C), jnp.float32)],
    )(vnbr, v, surf_dirs, conv_w, conv_b)


def _s3_body(vnbr_ref, vb_ref, cd_ref, cen_ref, supg_ref,
             f1w_ref, f1b_ref, wq_ref, wk_ref, wv_ref,
             pre_ref, q_ref, kv_ref):
    dirn = _dirn_from(vnbr_ref, vb_ref)
    theta2 = jnp.maximum(_mm(dirn, _coldir(cd_ref)), 0.0)
    act = theta2.reshape(_NN, _R, _C) * supg_ref[0]
    pre = cen_ref[0] + jnp.max(act, axis=0)
    x = _mm(pre, f1w_ref[...]) + f1b_ref[...]
    pre_ref[0] = pre
    q_ref[0] = _mm(x, wq_ref[...])
    kv_ref[0] = jnp.concatenate([_mm(x, wk_ref[...]), _mm(x, wv_ref[...])],
                                axis=1)


def _s3(vnbr, v, conv_dirs, center, supg, fc1_w, fc1_b, wq, wk, wv):
    return pl.pallas_call(
        _s3_body,
        grid=(_BS, _N // _R),
        in_specs=[
            pl.BlockSpec((1, _K21, _R, _VD), lambda b, r: (b, 0, r, 0)),
            pl.BlockSpec((1, _R, _VD), lambda b, r: (b, r, 0)),
            pl.BlockSpec((_VD, _C), lambda b, r: (0, 0)),
            pl.BlockSpec((1, _R, _C), lambda b, r: (b, r, 0)),
            pl.BlockSpec((1, _NN, _R, _C), lambda b, r: (b, 0, r, 0)),
            pl.BlockSpec((_C, _C), lambda b, r: (0, 0)),
            pl.BlockSpec((1, _C), lambda b, r: (0, 0)),
            pl.BlockSpec((_C, _C), lambda b, r: (0, 0)),
            pl.BlockSpec((_C, _C), lambda b, r: (0, 0)),
            pl.BlockSpec((_C, _C), lambda b, r: (0, 0)),
        ],
        out_specs=[pl.BlockSpec((1, _R, _C), lambda b, r: (b, r, 0)),
                   pl.BlockSpec((1, _R, _C), lambda b, r: (b, r, 0)),
                   pl.BlockSpec((1, _R, 2 * _C), lambda b, r: (b, r, 0))],
        out_shape=[jax.ShapeDtypeStruct((_BS, _N, _C), jnp.float32),
                   jax.ShapeDtypeStruct((_BS, _N, _C), jnp.float32),
                   jax.ShapeDtypeStruct((_BS, _N, 2 * _C), jnp.float32)],
    )(vnbr, v, conv_dirs, center, supg, fc1_w, fc1_b, wq, wk, wv)


def _s4_body(vnbr_ref, vb_ref, q_ref, kvg_ref, pre_ref,
             d1w_ref, d1b_ref, d2w_ref, d2b_ref,
             g1w_ref, g1b_ref, g2w_ref, g2b_ref,
             f2w_ref, f2b_ref, out_ref):
    vknn = vnbr_ref[0][:_AK]
    vb = vb_ref[0]
    delta = (vb[None, :, :] - vknn).reshape(_AK * _R, _VD)
    pos = _mm(jnp.maximum(_mm(delta, d1w_ref[...]) + d1b_ref[...], 0.0),
              d2w_ref[...]) + d2b_ref[...]
    kvg = kvg_ref[0]
    kk = kvg[:, :, :_C].reshape(_AK * _R, _C)
    vv = kvg[:, :, _C:].reshape(_AK * _R, _C)
    qb = jnp.broadcast_to(q_ref[0][None, :, :], (_AK, _R, _C))
    t = qb.reshape(_AK * _R, _C) - kk + pos
    attn = _mm(jnp.maximum(_mm(t, g1w_ref[...]) + g1b_ref[...], 0.0),
               g2w_ref[...]) + g2b_ref[...]
    attn = (attn * np.float32(1.0 / np.sqrt(_C))).reshape(_AK, _R, _C)
    attn = attn - jnp.max(attn, axis=0, keepdims=True)
    e = jnp.exp(attn)
    w = e / jnp.sum(e, axis=0, keepdims=True)
    res = jnp.sum(w * (vv + pos).reshape(_AK, _R, _C), axis=0)
    out_ref[0] = _mm(res, f2w_ref[...]) + f2b_ref[...] + pre_ref[0]


def _s4(vnbr, v, q, kvg, pre, d1_w, d1_b, d2_w, d2_b,
        g1_w, g1_b, g2_w, g2_b, fc2_w, fc2_b):
    wspec = lambda shape: pl.BlockSpec(shape, lambda b, r: tuple(0 for _ in shape))
    return pl.pallas_call(
        _s4_body,
        grid=(_BS, _N // _R),
        in_specs=[
            pl.BlockSpec((1, _K21, _R, _VD), lambda b, r: (b, 0, r, 0)),
            pl.BlockSpec((1, _R, _VD), lambda b, r: (b, r, 0)),
            pl.BlockSpec((1, _R, _C), lambda b, r: (b, r, 0)),
            pl.BlockSpec((1, _AK, _R, 2 * _C), lambda b, r: (b, 0, r, 0)),
            pl.BlockSpec((1, _R, _C), lambda b, r: (b, r, 0)),
            wspec((_VD, _C)), wspec((1, _C)), wspec((_C, _C)), wspec((1, _C)),
            wspec((_C, _C)), wspec((1, _C)), wspec((_C, _C)), wspec((1, _C)),
            wspec((_C, _C)), wspec((1, _C)),
        ],
        out_specs=pl.BlockSpec((1, _R, _C), lambda b, r: (b, r, 0)),
        out_shape=jax.ShapeDtypeStruct((_BS, _N, _C), jnp.float32),
    )(vnbr, v, q, kvg, pre, d1_w, d1_b, d2_w, d2_b,
      g1_w, g1_b, g2_w, g2_b, fc2_w, fc2_b)


def kernel(xyz, surf_dirs, conv_w, conv_b, conv_dirs, fc1_w, fc1_b, fc2_w,
           fc2_b, d1_w, d1_b, d2_w, d2_b, g1_w, g1_b, g2_w, g2_b, wq, wk, wv):
    v = jnp.transpose(xyz, (0, 2, 1))
    vpad = jnp.pad(v, ((0, 0), (0, 0), (0, _VD - 3)))
    sdp = jnp.pad(surf_dirs, ((0, _VD - 3), (0, 0)))
    cdp = jnp.pad(conv_dirs, ((0, _VD - 3), (0, 0)))
    d1p = jnp.pad(d1_w, ((0, _VD - 3), (0, 0)))

    gidx = _topk(vpad)
    gidx_t = jnp.transpose(gidx, (0, 2, 1))

    vnbr = _sc_gather(vpad.reshape(_BS * _N, _VD),
                      gidx_t.reshape(-1)).reshape(_BS, _K21, _N, _VD)

    center, support = _s2(vnbr, vpad, sdp, conv_w,
                          conv_b.reshape(1, 2 * _C))

    supg = _sc_gather(support.reshape(_BS * _N, _C),
                      gidx_t[:, 1:].reshape(-1)).reshape(_BS, _NN, _N, _C)

    pre, q, kv = _s3(vnbr, vpad, cdp, center, supg,
                     fc1_w, fc1_b.reshape(1, _C), wq, wk, wv)

    kvg = _sc_gather(kv.reshape(_BS * _N, 2 * _C),
                     gidx_t[:, :_AK].reshape(-1)).reshape(_BS, _AK, _N, 2 * _C)

    res = _s4(vnbr, vpad, q, kvg, pre,
              d1p, d1_b.reshape(1, _C), d2_w, d2_b.reshape(1, _C),
              g1_w, g1_b.reshape(1, _C), g2_w, g2_b.reshape(1, _C),
              fc2_w, fc2_b.reshape(1, _C))
    return jnp.transpose(res, (0, 2, 1))


# speedup vs baseline: 18.1276x; 1.3068x over previous
import functools

import numpy as np
import jax
import jax.numpy as jnp
from jax import lax
from jax.experimental import pallas as pl
from jax.experimental.pallas import tpu as pltpu
from jax.experimental.pallas import tpu_sc as plsc

_BS, _N, _NN, _AK, _C = 2, 4096, 20, 16, 128
_K21 = _NN + 1
_R = 256
_CH = 128
_NW = 32
_VD = 128

_T = 6
_NG = _N // 128


def _topk_body(vb_ref, va_ref, out_ref, d_scr):
    vb = vb_ref[...]
    va = va_ref[...]
    inner = lax.dot_general(vb, va, (((1,), (1,)), ((), ())),
                            preferred_element_type=jnp.float32)
    qb = jnp.sum(vb * vb, axis=1)
    qa = jnp.sum(va * va, axis=1)
    d_scr[...] = -2.0 * inner + qb[:, None] + qa[None, :]
    inf = jnp.float32(jnp.inf)
    bigi = jnp.int32(2**31 - 1)
    lane = lax.broadcasted_iota(jnp.int32, (_R, 128), 1)
    k_iota = lax.broadcasted_iota(jnp.int32, (_R, _K21), 1)

    ms = [jnp.full((_R, 128), inf, jnp.float32) for _ in range(_T)]
    cs = [jnp.full((_R, 128), bigi, jnp.int32) for _ in range(_T)]
    for g in range(_NG):
        d = d_scr[:, g * 128:(g + 1) * 128]
        c = lane + jnp.int32(g * 128)
        for t in range(_T):
            lt = d < ms[t]
            d2 = jnp.where(lt, ms[t], d)
            c2 = jnp.where(lt, cs[t], c)
            ms[t] = jnp.where(lt, d, ms[t])
            cs[t] = jnp.where(lt, c, cs[t])
            d, c = d2, c2

    cmat = jnp.concatenate(ms, axis=1)
    cidx = jnp.concatenate(cs, axis=1)

    def body(i, carry):
        cm, ci, idxacc, cnt = carry
        m = jnp.min(cm, axis=1, keepdims=True)
        sel = jnp.where(cm == m, ci, bigi)
        j = jnp.min(sel, axis=1, keepdims=True)
        cm = jnp.where((cm == m) & (ci == j), inf, cm)
        idxacc = jnp.where(k_iota == i, j, idxacc)
        cnt = cnt + jnp.where(lane == (j - (j // 128) * 128), 1, 0)
        return cm, ci, idxacc, cnt

    _, _, idxacc, cnt = lax.fori_loop(
        0, _K21, body,
        (cmat, cidx, jnp.zeros((_R, _K21), jnp.int32),
         jnp.zeros((_R, 128), jnp.int32)))

    need_exact = jnp.max(cnt) >= _T
    out_ref[...] = idxacc

    @pl.when(need_exact)
    def _fallback():
        cols = lax.broadcasted_iota(jnp.int32, (_R, _N), 1)

        def fbody(i, idxacc2):
            dmat = d_scr[...]
            m = jnp.min(dmat, axis=1, keepdims=True)
            sel = jnp.where(dmat == m, cols, bigi)
            j = jnp.min(sel, axis=1, keepdims=True)
            d_scr[...] = jnp.where(cols == j, inf, dmat)
            return jnp.where(k_iota == i, j, idxacc2)

        idxacc2 = lax.fori_loop(0, _K21, fbody,
                                jnp.zeros((_R, _K21), jnp.int32))
        out_ref[...] = idxacc2


def _topk(vpad2):
    return pl.pallas_call(
        _topk_body,
        grid=(_N // _R,),
        in_specs=[pl.BlockSpec((_R, _VD), lambda r: (r, 0)),
                  pl.BlockSpec((_N, _VD), lambda r: (0, 0))],
        out_specs=pl.BlockSpec((_R, _K21), lambda r: (r, 0)),
        out_shape=jax.ShapeDtypeStruct((_N, _K21), jnp.int32),
        scratch_shapes=[pltpu.VMEM((_R, _N), jnp.float32)],
    )(vpad2, vpad2)


def _sc_gather(table, idx):
    m_total = idx.shape[0]
    d = table.shape[1]
    nchunk = m_total // (_NW * _CH)
    idx2 = idx.reshape(_NW, nchunk * _CH)
    mesh = plsc.VectorSubcoreMesh(core_axis_name="c", subcore_axis_name="s")

    @functools.partial(
        pl.kernel,
        out_type=jax.ShapeDtypeStruct((m_total, d), jnp.float32),
        mesh=mesh,
        scratch_types=[
            pltpu.VMEM((nchunk * _CH,), jnp.int32),
            pltpu.VMEM((_CH, d), jnp.float32),
            pltpu.VMEM((_CH, d), jnp.float32),
            pltpu.SemaphoreType.DMA,
            pltpu.SemaphoreType.DMA,
        ],
    )
    def k(table_hbm, idx_hbm, out_hbm, idx_v, buf0, buf1, sem0, sem1):
        wid = lax.axis_index("s") * 2 + lax.axis_index("c")
        pltpu.sync_copy(idx_hbm.at[wid], idx_v)
        base = wid * (nchunk * _CH)
        bufs = (buf0, buf1)
        sems = (sem0, sem1)

        def start(j, bb):
            rows = idx_v.at[pl.ds(j * _CH, _CH)]
            pltpu.async_copy(table_hbm.at[rows], bufs[bb], sems[bb])

        def drain(bb):
            pltpu.make_async_copy(table_hbm.at[pl.ds(0, _CH)],
                                  bufs[bb], sems[bb]).wait()

        def emit(j, bb):
            drain(bb)
            pltpu.sync_copy(bufs[bb], out_hbm.at[pl.ds(base + j * _CH, _CH)])

        start(0, 0)
        if nchunk > 1:
            start(1, 1)

        def body(jj, carry):
            for bb in range(2):
                j = jj * 2 + bb
                emit(j, bb)

                @pl.when(j + 2 < nchunk)
                def _():
                    start(j + 2, bb)
            return carry

        lax.fori_loop(0, nchunk // 2, body, 0)
        if nchunk % 2:
            emit(nchunk - 1, (nchunk - 1) % 2)

    return k(table, idx2)


def _dirn_from(vnbr_ref, vb_ref):
    vn = vnbr_ref[...][1:]
    vb = vb_ref[...]
    diff = (vn - vb[None, :, :]).reshape(_NN * _R, _VD)
    ss = _mm(diff * diff, jnp.ones((_VD, _VD), jnp.float32))
    return diff / jnp.maximum(jnp.sqrt(ss), 1e-12)


def _coldir(w_ref):
    w = w_ref[...]
    nrm = jnp.sqrt(jnp.sum(w * w, axis=0, keepdims=True))
    return w / jnp.maximum(nrm, 1e-12)


def _mm(a, b):
    return lax.dot_general(a, b, (((1,), (0,)), ((), ())),
                           preferred_element_type=jnp.float32)


def _s2_body(vnbr_ref, vb_ref, sd_ref, cw_ref, cb_ref, cen_ref, sup_ref):
    dirn = _dirn_from(vnbr_ref, vb_ref)
    theta = jnp.maximum(_mm(dirn, _coldir(sd_ref)), 0.0)
    fea = jnp.maximum(jnp.max(theta.reshape(_NN, _R, _C), axis=0), 0.0)
    fout = _mm(fea, cw_ref[...]) + cb_ref[...]
    cen_ref[...] = fout[:, :_C]
    sup_ref[...] = fout[:, _C:]


def _s2(vnbr, vp, sdp, conv_w, conv_b):
    return pl.pallas_call(
        _s2_body,
        grid=(_N // _R,),
        in_specs=[
            pl.BlockSpec((_K21, _R, _VD), lambda r: (0, r, 0)),
            pl.BlockSpec((_R, _VD), lambda r: (r, 0)),
            pl.BlockSpec((_VD, _C), lambda r: (0, 0)),
            pl.BlockSpec((_C, 2 * _C), lambda r: (0, 0)),
            pl.BlockSpec((1, 2 * _C), lambda r: (0, 0)),
        ],
        out_specs=[pl.BlockSpec((_R, _C), lambda r: (r, 0)),
                   pl.BlockSpec((_R, _C), lambda r: (r, 0))],
        out_shape=[jax.ShapeDtypeStruct((_N, _C), jnp.float32),
                   jax.ShapeDtypeStruct((_N, _C), jnp.float32)],
    )(vnbr, vp, sdp, conv_w, conv_b)


def _s3_body(vnbr_ref, vb_ref, cd_ref, cen_ref, supg_ref,
             f1w_ref, f1b_ref, wq_ref, wk_ref, wv_ref,
             pre_ref, q_ref, kv_ref):
    dirn = _dirn_from(vnbr_ref, vb_ref)
    theta2 = jnp.maximum(_mm(dirn, _coldir(cd_ref)), 0.0)
    act = theta2.reshape(_NN, _R, _C) * supg_ref[...]
    pre = cen_ref[...] + jnp.max(act, axis=0)
    x = _mm(pre, f1w_ref[...]) + f1b_ref[...]
    pre_ref[...] = pre
    q_ref[...] = _mm(x, wq_ref[...])
    kv_ref[...] = jnp.concatenate([_mm(x, wk_ref[...]), _mm(x, wv_ref[...])],
                                  axis=1)


def _s3(vnbr, vp, cdp, center, supg, fc1_w, fc1_b, wq, wk, wv):
    return pl.pallas_call(
        _s3_body,
        grid=(_N // _R,),
        in_specs=[
            pl.BlockSpec((_K21, _R, _VD), lambda r: (0, r, 0)),
            pl.BlockSpec((_R, _VD), lambda r: (r, 0)),
            pl.BlockSpec((_VD, _C), lambda r: (0, 0)),
            pl.BlockSpec((_R, _C), lambda r: (r, 0)),
            pl.BlockSpec((_NN, _R, _C), lambda r: (0, r, 0)),
            pl.BlockSpec((_C, _C), lambda r: (0, 0)),
            pl.BlockSpec((1, _C), lambda r: (0, 0)),
            pl.BlockSpec((_C, _C), lambda r: (0, 0)),
            pl.BlockSpec((_C, _C), lambda r: (0, 0)),
            pl.BlockSpec((_C, _C), lambda r: (0, 0)),
        ],
        out_specs=[pl.BlockSpec((_R, _C), lambda r: (r, 0)),
                   pl.BlockSpec((_R, _C), lambda r: (r, 0)),
                   pl.BlockSpec((_R, 2 * _C), lambda r: (r, 0))],
        out_shape=[jax.ShapeDtypeStruct((_N, _C), jnp.float32),
                   jax.ShapeDtypeStruct((_N, _C), jnp.float32),
                   jax.ShapeDtypeStruct((_N, 2 * _C), jnp.float32)],
    )(vnbr, vp, cdp, center, supg, fc1_w, fc1_b, wq, wk, wv)


def _s4_body(vnbr_ref, vb_ref, q_ref, kvg_ref, pre_ref,
             d1w_ref, d1b_ref, d2w_ref, d2b_ref,
             g1w_ref, g1b_ref, g2w_ref, g2b_ref,
             f2w_ref, f2b_ref, out_ref):
    vknn = vnbr_ref[...][:_AK]
    vb = vb_ref[...]
    delta = (vb[None, :, :] - vknn).reshape(_AK * _R, _VD)
    pos = _mm(jnp.maximum(_mm(delta, d1w_ref[...]) + d1b_ref[...], 0.0),
              d2w_ref[...]) + d2b_ref[...]
    kvg = kvg_ref[...]
    kk = kvg[:, :, :_C].reshape(_AK * _R, _C)
    vv = kvg[:, :, _C:].reshape(_AK * _R, _C)
    qb = jnp.broadcast_to(q_ref[...][None, :, :], (_AK, _R, _C))
    t = qb.reshape(_AK * _R, _C) - kk + pos
    attn = _mm(jnp.maximum(_mm(t, g1w_ref[...]) + g1b_ref[...], 0.0),
               g2w_ref[...]) + g2b_ref[...]
    attn = (attn * np.float32(1.0 / np.sqrt(_C))).reshape(_AK, _R, _C)
    attn = attn - jnp.max(attn, axis=0, keepdims=True)
    e = jnp.exp(attn)
    w = e / jnp.sum(e, axis=0, keepdims=True)
    res = jnp.sum(w * (vv + pos).reshape(_AK, _R, _C), axis=0)
    out_ref[...] = _mm(res, f2w_ref[...]) + f2b_ref[...] + pre_ref[...]


def _s4(vnbr, vp, q, kvg, pre, d1p, d1_b, d2_w, d2_b,
        g1_w, g1_b, g2_w, g2_b, fc2_w, fc2_b):
    wspec = lambda shape: pl.BlockSpec(shape, lambda r: tuple(0 for _ in shape))
    return pl.pallas_call(
        _s4_body,
        grid=(_N // _R,),
        in_specs=[
            pl.BlockSpec((_K21, _R, _VD), lambda r: (0, r, 0)),
            pl.BlockSpec((_R, _VD), lambda r: (r, 0)),
            pl.BlockSpec((_R, _C), lambda r: (r, 0)),
            pl.BlockSpec((_AK, _R, 2 * _C), lambda r: (0, r, 0)),
            pl.BlockSpec((_R, _C), lambda r: (r, 0)),
            wspec((_VD, _C)), wspec((1, _C)), wspec((_C, _C)), wspec((1, _C)),
            wspec((_C, _C)), wspec((1, _C)), wspec((_C, _C)), wspec((1, _C)),
            wspec((_C, _C)), wspec((1, _C)),
        ],
        out_specs=pl.BlockSpec((_R, _C), lambda r: (r, 0)),
        out_shape=jax.ShapeDtypeStruct((_N, _C), jnp.float32),
    )(vnbr, vp, q, kvg, pre, d1p, d1_b, d2_w, d2_b,
      g1_w, g1_b, g2_w, g2_b, fc2_w, fc2_b)


def kernel(xyz, surf_dirs, conv_w, conv_b, conv_dirs, fc1_w, fc1_b, fc2_w,
           fc2_b, d1_w, d1_b, d2_w, d2_b, g1_w, g1_b, g2_w, g2_b, wq, wk, wv):
    v = jnp.transpose(xyz, (0, 2, 1))
    vpad = jnp.pad(v, ((0, 0), (0, 0), (0, _VD - 3)))
    sdp = jnp.pad(surf_dirs, ((0, _VD - 3), (0, 0)))
    cdp = jnp.pad(conv_dirs, ((0, _VD - 3), (0, 0)))
    d1p = jnp.pad(d1_w, ((0, _VD - 3), (0, 0)))
    cb2 = conv_b.reshape(1, 2 * _C)
    f1b2 = fc1_b.reshape(1, _C)
    d1b2 = d1_b.reshape(1, _C)
    d2b2 = d2_b.reshape(1, _C)
    g1b2 = g1_b.reshape(1, _C)
    g2b2 = g2_b.reshape(1, _C)
    f2b2 = fc2_b.reshape(1, _C)

    outs = []
    for b in range(_BS):
        vp = vpad[b]
        gidx = _topk(vp)
        gidx_t = jnp.transpose(gidx, (1, 0))

        vnbr = _sc_gather(vp, gidx_t.reshape(-1)).reshape(_K21, _N, _VD)
        center, support = _s2(vnbr, vp, sdp, conv_w, cb2)
        supg = _sc_gather(support,
                          gidx_t[1:].reshape(-1)).reshape(_NN, _N, _C)
        pre, q, kv = _s3(vnbr, vp, cdp, center, supg, fc1_w, f1b2, wq, wk, wv)
        kvg = _sc_gather(kv,
                         gidx_t[:_AK].reshape(-1)).reshape(_AK, _N, 2 * _C)
        outs.append(_s4(vnbr, vp, q, kvg, pre, d1p, d1b2, d2_w, d2b2,
                        g1_w, g1b2, g2_w, g2b2, fc2_w, f2b2))

    return jnp.transpose(jnp.stack(outs), (0, 2, 1))


# speedup vs baseline: 18.1571x; 1.0016x over previous
import functools

import numpy as np
import jax
import jax.numpy as jnp
from jax import lax
from jax.experimental import pallas as pl
from jax.experimental.pallas import tpu as pltpu
from jax.experimental.pallas import tpu_sc as plsc

_BS, _N, _NN, _AK, _C = 2, 4096, 20, 16, 128
_K21 = _NN + 1
_R = 256
_CH = 128
_NW = 32
_VD = 128

_T = 6
_NG = _N // 128


def _topk_body(vb_ref, va_ref, out_ref, d_scr):
    vb = vb_ref[...]
    va = va_ref[...]
    inner = lax.dot_general(vb, va, (((1,), (1,)), ((), ())),
                            preferred_element_type=jnp.float32)
    qb = jnp.sum(vb * vb, axis=1)
    qa = jnp.sum(va * va, axis=1)
    d_scr[...] = -2.0 * inner + qb[:, None] + qa[None, :]
    inf = jnp.float32(jnp.inf)
    bigi = jnp.int32(2**31 - 1)
    lane = lax.broadcasted_iota(jnp.int32, (_R, 128), 1)
    k_iota = lax.broadcasted_iota(jnp.int32, (_R, _K21), 1)

    ms = [jnp.full((_R, 128), inf, jnp.float32) for _ in range(_T)]
    cs = [jnp.full((_R, 128), bigi, jnp.int32) for _ in range(_T)]
    for g in range(_NG):
        d = d_scr[:, g * 128:(g + 1) * 128]
        c = lane + jnp.int32(g * 128)
        for t in range(_T):
            lt = d < ms[t]
            d2 = jnp.where(lt, ms[t], d)
            c2 = jnp.where(lt, cs[t], c)
            ms[t] = jnp.where(lt, d, ms[t])
            cs[t] = jnp.where(lt, c, cs[t])
            d, c = d2, c2

    cmat = jnp.concatenate(ms, axis=1)
    cidx = jnp.concatenate(cs, axis=1)

    def body(i, carry):
        cm, ci, idxacc, cnt = carry
        m = jnp.min(cm, axis=1, keepdims=True)
        sel = jnp.where(cm == m, ci, bigi)
        j = jnp.min(sel, axis=1, keepdims=True)
        cm = jnp.where((cm == m) & (ci == j), inf, cm)
        idxacc = jnp.where(k_iota == i, j, idxacc)
        cnt = cnt + jnp.where(lane == (j - (j // 128) * 128), 1, 0)
        return cm, ci, idxacc, cnt

    _, _, idxacc, cnt = lax.fori_loop(
        0, _K21, body,
        (cmat, cidx, jnp.zeros((_R, _K21), jnp.int32),
         jnp.zeros((_R, 128), jnp.int32)))

    need_exact = jnp.max(cnt) >= _T
    out_ref[...] = idxacc

    @pl.when(need_exact)
    def _fallback():
        cols = lax.broadcasted_iota(jnp.int32, (_R, _N), 1)

        def fbody(i, idxacc2):
            dmat = d_scr[...]
            m = jnp.min(dmat, axis=1, keepdims=True)
            sel = jnp.where(dmat == m, cols, bigi)
            j = jnp.min(sel, axis=1, keepdims=True)
            d_scr[...] = jnp.where(cols == j, inf, dmat)
            return jnp.where(k_iota == i, j, idxacc2)

        idxacc2 = lax.fori_loop(0, _K21, fbody,
                                jnp.zeros((_R, _K21), jnp.int32))
        out_ref[...] = idxacc2


def _topk(vpad8):
    return pl.pallas_call(
        _topk_body,
        grid=(_N // _R,),
        in_specs=[pl.BlockSpec((_R, 8), lambda r: (r, 0)),
                  pl.BlockSpec((_N, 8), lambda r: (0, 0))],
        out_specs=pl.BlockSpec((_R, _K21), lambda r: (r, 0)),
        out_shape=jax.ShapeDtypeStruct((_N, _K21), jnp.int32),
        scratch_shapes=[pltpu.VMEM((_R, _N), jnp.float32)],
    )(vpad8, vpad8)


def _sc_gather(table, idx):
    m_total = idx.shape[0]
    d = table.shape[1]
    nchunk = m_total // (_NW * _CH)
    idx2 = idx.reshape(_NW, nchunk * _CH)
    mesh = plsc.VectorSubcoreMesh(core_axis_name="c", subcore_axis_name="s")

    @functools.partial(
        pl.kernel,
        out_type=jax.ShapeDtypeStruct((m_total, d), jnp.float32),
        mesh=mesh,
        scratch_types=[
            pltpu.VMEM((nchunk * _CH,), jnp.int32),
            pltpu.VMEM((_CH, d), jnp.float32),
            pltpu.VMEM((_CH, d), jnp.float32),
            pltpu.SemaphoreType.DMA,
            pltpu.SemaphoreType.DMA,
        ],
    )
    def k(table_hbm, idx_hbm, out_hbm, idx_v, buf0, buf1, sem0, sem1):
        wid = lax.axis_index("s") * 2 + lax.axis_index("c")
        pltpu.sync_copy(idx_hbm.at[wid], idx_v)
        base = wid * (nchunk * _CH)
        bufs = (buf0, buf1)
        sems = (sem0, sem1)

        def start(j, bb):
            rows = idx_v.at[pl.ds(j * _CH, _CH)]
            pltpu.async_copy(table_hbm.at[rows], bufs[bb], sems[bb])

        def drain(bb):
            pltpu.make_async_copy(table_hbm.at[pl.ds(0, _CH)],
                                  bufs[bb], sems[bb]).wait()

        def emit(j, bb):
            drain(bb)
            pltpu.sync_copy(bufs[bb], out_hbm.at[pl.ds(base + j * _CH, _CH)])

        start(0, 0)
        if nchunk > 1:
            start(1, 1)

        def body(jj, carry):
            for bb in range(2):
                j = jj * 2 + bb
                emit(j, bb)

                @pl.when(j + 2 < nchunk)
                def _():
                    start(j + 2, bb)
            return carry

        lax.fori_loop(0, nchunk // 2, body, 0)
        if nchunk % 2:
            emit(nchunk - 1, (nchunk - 1) % 2)

    return k(table, idx2)


def _dirn_from(vnbr_ref, vb_ref):
    vn = vnbr_ref[...][1:]
    vb = vb_ref[...]
    diff = (vn - vb[None, :, :]).reshape(_NN * _R, _VD)
    ss = _mm(diff * diff, jnp.ones((_VD, _VD), jnp.float32))
    return diff / jnp.maximum(jnp.sqrt(ss), 1e-12)


def _coldir(w_ref):
    w = w_ref[...]
    nrm = jnp.sqrt(jnp.sum(w * w, axis=0, keepdims=True))
    return w / jnp.maximum(nrm, 1e-12)


def _mm(a, b):
    return lax.dot_general(a, b, (((1,), (0,)), ((), ())),
                           preferred_element_type=jnp.float32)


def _s2_body(vnbr_ref, vb_ref, sd_ref, cw_ref, cb_ref, cen_ref, sup_ref):
    dirn = _dirn_from(vnbr_ref, vb_ref)
    theta = jnp.maximum(_mm(dirn, _coldir(sd_ref)), 0.0)
    fea = jnp.maximum(jnp.max(theta.reshape(_NN, _R, _C), axis=0), 0.0)
    fout = _mm(fea, cw_ref[...]) + cb_ref[...]
    cen_ref[...] = fout[:, :_C]
    sup_ref[...] = fout[:, _C:]


def _s2(vnbr, vp, sdp, conv_w, conv_b):
    return pl.pallas_call(
        _s2_body,
        grid=(_N // _R,),
        in_specs=[
            pl.BlockSpec((_K21, _R, _VD), lambda r: (0, r, 0)),
            pl.BlockSpec((_R, _VD), lambda r: (r, 0)),
            pl.BlockSpec((_VD, _C), lambda r: (0, 0)),
            pl.BlockSpec((_C, 2 * _C), lambda r: (0, 0)),
            pl.BlockSpec((1, 2 * _C), lambda r: (0, 0)),
        ],
        out_specs=[pl.BlockSpec((_R, _C), lambda r: (r, 0)),
                   pl.BlockSpec((_R, _C), lambda r: (r, 0))],
        out_shape=[jax.ShapeDtypeStruct((_N, _C), jnp.float32),
                   jax.ShapeDtypeStruct((_N, _C), jnp.float32)],
    )(vnbr, vp, sdp, conv_w, conv_b)


def _s3_body(vnbr_ref, vb_ref, cd_ref, cen_ref, supg_ref,
             f1w_ref, f1b_ref, wq_ref, wk_ref, wv_ref,
             pre_ref, q_ref, kv_ref):
    dirn = _dirn_from(vnbr_ref, vb_ref)
    theta2 = jnp.maximum(_mm(dirn, _coldir(cd_ref)), 0.0)
    act = theta2.reshape(_NN, _R, _C) * supg_ref[...]
    pre = cen_ref[...] + jnp.max(act, axis=0)
    x = _mm(pre, f1w_ref[...]) + f1b_ref[...]
    pre_ref[...] = pre
    q_ref[...] = _mm(x, wq_ref[...])
    kv_ref[...] = jnp.concatenate([_mm(x, wk_ref[...]), _mm(x, wv_ref[...])],
                                  axis=1)


def _s3(vnbr, vp, cdp, center, supg, fc1_w, fc1_b, wq, wk, wv):
    return pl.pallas_call(
        _s3_body,
        grid=(_N // _R,),
        in_specs=[
            pl.BlockSpec((_K21, _R, _VD), lambda r: (0, r, 0)),
            pl.BlockSpec((_R, _VD), lambda r: (r, 0)),
            pl.BlockSpec((_VD, _C), lambda r: (0, 0)),
            pl.BlockSpec((_R, _C), lambda r: (r, 0)),
            pl.BlockSpec((_NN, _R, _C), lambda r: (0, r, 0)),
            pl.BlockSpec((_C, _C), lambda r: (0, 0)),
            pl.BlockSpec((1, _C), lambda r: (0, 0)),
            pl.BlockSpec((_C, _C), lambda r: (0, 0)),
            pl.BlockSpec((_C, _C), lambda r: (0, 0)),
            pl.BlockSpec((_C, _C), lambda r: (0, 0)),
        ],
        out_specs=[pl.BlockSpec((_R, _C), lambda r: (r, 0)),
                   pl.BlockSpec((_R, _C), lambda r: (r, 0)),
                   pl.BlockSpec((_R, 2 * _C), lambda r: (r, 0))],
        out_shape=[jax.ShapeDtypeStruct((_N, _C), jnp.float32),
                   jax.ShapeDtypeStruct((_N, _C), jnp.float32),
                   jax.ShapeDtypeStruct((_N, 2 * _C), jnp.float32)],
    )(vnbr, vp, cdp, center, supg, fc1_w, fc1_b, wq, wk, wv)


def _s4_body(vnbr_ref, vb_ref, q_ref, kvg_ref, pre_ref,
             d1w_ref, d1b_ref, d2w_ref, d2b_ref,
             g1w_ref, g1b_ref, g2w_ref, g2b_ref,
             f2w_ref, f2b_ref, out_ref):
    vknn = vnbr_ref[...][:_AK]
    vb = vb_ref[...]
    delta = (vb[None, :, :] - vknn).reshape(_AK * _R, _VD)
    pos = _mm(jnp.maximum(_mm(delta, d1w_ref[...]) + d1b_ref[...], 0.0),
              d2w_ref[...]) + d2b_ref[...]
    kvg = kvg_ref[...]
    kk = kvg[:, :, :_C].reshape(_AK * _R, _C)
    vv = kvg[:, :, _C:].reshape(_AK * _R, _C)
    qb = jnp.broadcast_to(q_ref[...][None, :, :], (_AK, _R, _C))
    t = qb.reshape(_AK * _R, _C) - kk + pos
    attn = _mm(jnp.maximum(_mm(t, g1w_ref[...]) + g1b_ref[...], 0.0),
               g2w_ref[...]) + g2b_ref[...]
    attn = (attn * np.float32(1.0 / np.sqrt(_C))).reshape(_AK, _R, _C)
    attn = attn - jnp.max(attn, axis=0, keepdims=True)
    e = jnp.exp(attn)
    w = e / jnp.sum(e, axis=0, keepdims=True)
    res = jnp.sum(w * (vv + pos).reshape(_AK, _R, _C), axis=0)
    out_ref[...] = _mm(res, f2w_ref[...]) + f2b_ref[...] + pre_ref[...]


def _s4(vnbr, vp, q, kvg, pre, d1p, d1_b, d2_w, d2_b,
        g1_w, g1_b, g2_w, g2_b, fc2_w, fc2_b):
    wspec = lambda shape: pl.BlockSpec(shape, lambda r: tuple(0 for _ in shape))
    return pl.pallas_call(
        _s4_body,
        grid=(_N // _R,),
        in_specs=[
            pl.BlockSpec((_K21, _R, _VD), lambda r: (0, r, 0)),
            pl.BlockSpec((_R, _VD), lambda r: (r, 0)),
            pl.BlockSpec((_R, _C), lambda r: (r, 0)),
            pl.BlockSpec((_AK, _R, 2 * _C), lambda r: (0, r, 0)),
            pl.BlockSpec((_R, _C), lambda r: (r, 0)),
            wspec((_VD, _C)), wspec((1, _C)), wspec((_C, _C)), wspec((1, _C)),
            wspec((_C, _C)), wspec((1, _C)), wspec((_C, _C)), wspec((1, _C)),
            wspec((_C, _C)), wspec((1, _C)),
        ],
        out_specs=pl.BlockSpec((_R, _C), lambda r: (r, 0)),
        out_shape=jax.ShapeDtypeStruct((_N, _C), jnp.float32),
    )(vnbr, vp, q, kvg, pre, d1p, d1_b, d2_w, d2_b,
      g1_w, g1_b, g2_w, g2_b, fc2_w, fc2_b)


def kernel(xyz, surf_dirs, conv_w, conv_b, conv_dirs, fc1_w, fc1_b, fc2_w,
           fc2_b, d1_w, d1_b, d2_w, d2_b, g1_w, g1_b, g2_w, g2_b, wq, wk, wv):
    v = jnp.transpose(xyz, (0, 2, 1))
    vpad = jnp.pad(v, ((0, 0), (0, 0), (0, _VD - 3)))
    vpad8 = jnp.pad(v, ((0, 0), (0, 0), (0, 5)))
    sdp = jnp.pad(surf_dirs, ((0, _VD - 3), (0, 0)))
    cdp = jnp.pad(conv_dirs, ((0, _VD - 3), (0, 0)))
    d1p = jnp.pad(d1_w, ((0, _VD - 3), (0, 0)))
    cb2 = conv_b.reshape(1, 2 * _C)
    f1b2 = fc1_b.reshape(1, _C)
    d1b2 = d1_b.reshape(1, _C)
    d2b2 = d2_b.reshape(1, _C)
    g1b2 = g1_b.reshape(1, _C)
    g2b2 = g2_b.reshape(1, _C)
    f2b2 = fc2_b.reshape(1, _C)

    outs = []
    for b in range(_BS):
        vp = vpad[b]
        gidx = _topk(vpad8[b])
        gidx_t = jnp.transpose(gidx, (1, 0))

        vnbr = _sc_gather(vp, gidx_t.reshape(-1)).reshape(_K21, _N, _VD)
        center, support = _s2(vnbr, vp, sdp, conv_w, cb2)
        supg = _sc_gather(support,
                          gidx_t[1:].reshape(-1)).reshape(_NN, _N, _C)
        pre, q, kv = _s3(vnbr, vp, cdp, center, supg, fc1_w, f1b2, wq, wk, wv)
        kvg = _sc_gather(kv,
                         gidx_t[:_AK].reshape(-1)).reshape(_AK, _N, 2 * _C)
        outs.append(_s4(vnbr, vp, q, kvg, pre, d1p, d1b2, d2_w, d2b2,
                        g1_w, g1b2, g2_w, g2b2, fc2_w, f2b2))

    return jnp.transpose(jnp.stack(outs), (0, 2, 1))
